# baseline (speedup 1.0000x reference)
import jax
import jax.numpy as jnp
from jax.experimental import pallas as pl


def _cell_body(tmap_ref, nh_ref, nc_ref, fin_ref, ufc_ref, uiouc_ref,
               bfr_ref, biour_ref, iou_ref, c_ref):
    BN, K, H = nh_ref.shape
    NT = bfr_ref.shape[1] // H
    nh = nh_ref[...]
    nc = nc_ref[...]
    fin = fin_ref[...]
    tmap = tmap_ref[...]

    h_tilde = jnp.sum(nh, axis=1)

    piou = jnp.dot(h_tilde.astype(jnp.bfloat16), uiouc_ref[...],
                   preferred_element_type=jnp.float32)
    O = 3 * H
    t1 = tmap[:, :1]
    iou_sel = piou[:, 0:O]
    biou_sel = biour_ref[0:1, 0:O]
    for t in range(1, NT):
        cond = t1 == t
        iou_sel = jnp.where(cond, piou[:, t * O:(t + 1) * O], iou_sel)
        biou_sel = jnp.where(cond, biour_ref[0:1, t * O:(t + 1) * O],
                             biou_sel)
    iou_ref[...] = iou_sel + biou_sel

    pf = jnp.dot(nh.reshape(BN * K, H).astype(jnp.bfloat16), ufc_ref[...],
                 preferred_element_type=jnp.float32)
    pf = pf.reshape(BN, K, NT * H)
    cond3 = t1[:, :, None]
    f_sel = pf[:, :, 0:H]
    bf_sel = bfr_ref[0:1, 0:H]
    for t in range(1, NT):
        f_sel = jnp.where(cond3 == t, pf[:, :, t * H:(t + 1) * H], f_sel)
        bf_sel = jnp.where(t1 == t, bfr_ref[0:1, t * H:(t + 1) * H], bf_sel)
    neg_log2e = -1.4426950408889634
    ys = f_sel + ((fin + bf_sel) * neg_log2e)[:, None, :]
    gate = 1.0 / (1.0 + jnp.exp2(ys))
    c_ref[...] = jnp.sum(gate * nc, axis=1)


def kernel(n_h, n_c, f_in, type_id, U_iou, b_iou, U_f, b_f):
    N, K, H = n_h.shape
    NT = U_iou.shape[0]
    BN = 200
    nb = N // BN

    tmap = jnp.broadcast_to(type_id.astype(jnp.int32)[:, None], (N, 8))
    ufc = (U_f.transpose(1, 0, 2).reshape(H, NT * H)
           * (-1.4426950408889634)).astype(jnp.bfloat16)
    uiouc = U_iou.transpose(1, 0, 2).reshape(H, NT * 3 * H).astype(jnp.bfloat16)
    bfr = jnp.tile(b_f.reshape(1, NT * H), (8, 1))
    biour = jnp.tile(b_iou.reshape(1, NT * 3 * H), (8, 1))

    iou_aggr, c_aggr = pl.pallas_call(
        _cell_body,
        grid=(nb,),
        in_specs=[
            pl.BlockSpec((BN, 8), lambda i: (i, 0)),
            pl.BlockSpec((BN, K, H), lambda i: (i, 0, 0)),
            pl.BlockSpec((BN, K, H), lambda i: (i, 0, 0)),
            pl.BlockSpec((BN, H), lambda i: (i, 0)),
            pl.BlockSpec((H, NT * H), lambda i: (0, 0)),
            pl.BlockSpec((H, NT * 3 * H), lambda i: (0, 0)),
            pl.BlockSpec((8, NT * H), lambda i: (0, 0)),
            pl.BlockSpec((8, NT * 3 * H), lambda i: (0, 0)),
        ],
        out_specs=[
            pl.BlockSpec((BN, 3 * H), lambda i: (i, 0)),
            pl.BlockSpec((BN, H), lambda i: (i, 0)),
        ],
        out_shape=[
            jax.ShapeDtypeStruct((N, 3 * H), n_h.dtype),
            jax.ShapeDtypeStruct((N, H), n_h.dtype),
        ],
    )(tmap, n_h, n_c, f_in, ufc, uiouc, bfr, biour)
    return iou_aggr, c_aggr


# speedup vs baseline: 1.1036x; 1.1036x over previous
import jax
import jax.numpy as jnp
from jax.experimental import pallas as pl


def _cell_body(tmap_ref, nh_ref, nc_ref, fin_ref, ufc_ref, uiouc_ref,
               bfr_ref, biour_ref, iou_ref, c_ref):
    BN, K, H = nh_ref.shape
    NT = bfr_ref.shape[1] // H
    nh = nh_ref[...]
    nc = nc_ref[...]
    fin = fin_ref[...]
    tmap = tmap_ref[...]

    h_tilde = jnp.sum(nh, axis=1)

    piou = jnp.dot(h_tilde.astype(jnp.bfloat16), uiouc_ref[...],
                   preferred_element_type=jnp.float32)
    O = 3 * H
    t1 = tmap[:, :1]
    iou_sel = piou[:, 0:O]
    biou_sel = biour_ref[0:1, 0:O]
    for t in range(1, NT):
        cond = t1 == t
        iou_sel = jnp.where(cond, piou[:, t * O:(t + 1) * O], iou_sel)
        biou_sel = jnp.where(cond, biour_ref[0:1, t * O:(t + 1) * O],
                             biou_sel)
    iou_ref[...] = iou_sel + biou_sel

    pf = jnp.dot(nh.reshape(BN * K, H).astype(jnp.bfloat16), ufc_ref[...],
                 preferred_element_type=jnp.float32)
    pf = pf.reshape(BN, K, NT * H)
    tb = jnp.broadcast_to(tmap[:, :1], (BN, H))
    cond3 = tb[:, None, :]
    f_sel = pf[:, :, 0:H]
    bf_sel = bfr_ref[0:1, 0:H]
    for t in range(1, NT):
        f_sel = jnp.where((cond3 == t), pf[:, :, t * H:(t + 1) * H], f_sel)
        bf_sel = jnp.where(t1 == t, bfr_ref[0:1, t * H:(t + 1) * H], bf_sel)
    neg_log2e = -1.4426950408889634
    ys = f_sel + ((fin + bf_sel) * neg_log2e)[:, None, :]
    gate = 1.0 / (1.0 + jnp.exp2(ys))
    c_ref[...] = jnp.sum(gate * nc, axis=1)


def kernel(n_h, n_c, f_in, type_id, U_iou, b_iou, U_f, b_f):
    N, K, H = n_h.shape
    NT = U_iou.shape[0]
    BN = 200
    nb = N // BN

    tmap = jnp.broadcast_to(type_id.astype(jnp.int32)[:, None], (N, 8))
    ufc = (U_f.transpose(1, 0, 2).reshape(H, NT * H)
           * (-1.4426950408889634)).astype(jnp.bfloat16)
    uiouc = U_iou.transpose(1, 0, 2).reshape(H, NT * 3 * H).astype(jnp.bfloat16)
    bfr = jnp.tile(b_f.reshape(1, NT * H), (8, 1))
    biour = jnp.tile(b_iou.reshape(1, NT * 3 * H), (8, 1))

    iou_aggr, c_aggr = pl.pallas_call(
        _cell_body,
        grid=(nb,),
        in_specs=[
            pl.BlockSpec((BN, 8), lambda i: (i, 0)),
            pl.BlockSpec((BN, K, H), lambda i: (i, 0, 0)),
            pl.BlockSpec((BN, K, H), lambda i: (i, 0, 0)),
            pl.BlockSpec((BN, H), lambda i: (i, 0)),
            pl.BlockSpec((H, NT * H), lambda i: (0, 0)),
            pl.BlockSpec((H, NT * 3 * H), lambda i: (0, 0)),
            pl.BlockSpec((8, NT * H), lambda i: (0, 0)),
            pl.BlockSpec((8, NT * 3 * H), lambda i: (0, 0)),
        ],
        out_specs=[
            pl.BlockSpec((BN, 3 * H), lambda i: (i, 0)),
            pl.BlockSpec((BN, H), lambda i: (i, 0)),
        ],
        out_shape=[
            jax.ShapeDtypeStruct((N, 3 * H), n_h.dtype),
            jax.ShapeDtypeStruct((N, H), n_h.dtype),
        ],
    )(tmap, n_h, n_c, f_in, ufc, uiouc, bfr, biour)
    return iou_aggr, c_aggr


# speedup vs baseline: 1.2032x; 1.0903x over previous
import jax
import jax.numpy as jnp
from jax.experimental import pallas as pl


def _cell_body(tmap_ref, nh_ref, nc_ref, fin_ref, ufc_ref, uiouc_ref,
               bfr_ref, biour_ref, iou_ref, c_ref):
    BN, K, H = nh_ref.shape
    NT = bfr_ref.shape[1] // H
    nh = nh_ref[...]
    nc = nc_ref[...]
    fin = fin_ref[...]
    tmap = tmap_ref[...]

    h_tilde = jnp.sum(nh, axis=1)

    piou = jnp.dot(h_tilde.astype(jnp.bfloat16), uiouc_ref[...],
                   preferred_element_type=jnp.float32)
    O = 3 * H
    t1 = tmap[:, :1]
    iou_sel = piou[:, 0:O]
    biou_sel = biour_ref[0:1, 0:O]
    for t in range(1, NT):
        cond = t1 == t
        iou_sel = jnp.where(cond, piou[:, t * O:(t + 1) * O], iou_sel)
        biou_sel = jnp.where(cond, biour_ref[0:1, t * O:(t + 1) * O],
                             biou_sel)
    iou_ref[...] = iou_sel + biou_sel

    pf = jnp.dot(nh.reshape(BN * K, H).astype(jnp.bfloat16), ufc_ref[...],
                 preferred_element_type=jnp.float32)
    pf = pf.reshape(BN, K, NT * H)
    tb = jnp.broadcast_to(tmap[:, :1], (BN, H))
    cond3 = tb[:, None, :]
    f_sel = pf[:, :, 0:H]
    bf_sel = bfr_ref[0:1, 0:H]
    for t in range(1, NT):
        f_sel = jnp.where((cond3 == t), pf[:, :, t * H:(t + 1) * H], f_sel)
        bf_sel = jnp.where(t1 == t, bfr_ref[0:1, t * H:(t + 1) * H], bf_sel)
    neg_log2e = -1.4426950408889634
    ys = f_sel + ((fin + bf_sel) * neg_log2e)[:, None, :]
    gate = 1.0 / (1.0 + jnp.exp2(ys))
    c_ref[...] = jnp.sum(gate * nc, axis=1)


def kernel(n_h, n_c, f_in, type_id, U_iou, b_iou, U_f, b_f):
    N, K, H = n_h.shape
    NT = U_iou.shape[0]
    BN = 400
    nb = N // BN

    tmap = jnp.broadcast_to(type_id.astype(jnp.int32)[:, None], (N, 8))
    ufc = (U_f.transpose(1, 0, 2).reshape(H, NT * H)
           * (-1.4426950408889634)).astype(jnp.bfloat16)
    uiouc = U_iou.transpose(1, 0, 2).reshape(H, NT * 3 * H).astype(jnp.bfloat16)
    bfr = jnp.tile(b_f.reshape(1, NT * H), (8, 1))
    biour = jnp.tile(b_iou.reshape(1, NT * 3 * H), (8, 1))

    iou_aggr, c_aggr = pl.pallas_call(
        _cell_body,
        grid=(nb,),
        in_specs=[
            pl.BlockSpec((BN, 8), lambda i: (i, 0)),
            pl.BlockSpec((BN, K, H), lambda i: (i, 0, 0)),
            pl.BlockSpec((BN, K, H), lambda i: (i, 0, 0)),
            pl.BlockSpec((BN, H), lambda i: (i, 0)),
            pl.BlockSpec((H, NT * H), lambda i: (0, 0)),
            pl.BlockSpec((H, NT * 3 * H), lambda i: (0, 0)),
            pl.BlockSpec((8, NT * H), lambda i: (0, 0)),
            pl.BlockSpec((8, NT * 3 * H), lambda i: (0, 0)),
        ],
        out_specs=[
            pl.BlockSpec((BN, 3 * H), lambda i: (i, 0)),
            pl.BlockSpec((BN, H), lambda i: (i, 0)),
        ],
        out_shape=[
            jax.ShapeDtypeStruct((N, 3 * H), n_h.dtype),
            jax.ShapeDtypeStruct((N, H), n_h.dtype),
        ],
    )(tmap, n_h, n_c, f_in, ufc, uiouc, bfr, biour)
    return iou_aggr, c_aggr
